# baseline (speedup 1.0000x reference)
import functools

import jax
import jax.numpy as jnp
from jax import lax
from jax.experimental import pallas as pl
from jax.experimental.pallas import tpu as pltpu
from jax.experimental.pallas import tpu_sc as plsc

B, T, D, V, K_DB, TOPK = 16, 8, 1024, 100000, 65536, 32
N = B * T
KC = 2048
NKC = K_DB // KC
NG = K_DB // 128
VC = 8192
NVC = (V + VC - 1) // VC
NEG = float("-inf")


def _vstats_body(lg_ref, m_out, z_out, mrun, zrun):
    j = pl.program_id(0)

    @pl.when(j == 0)
    def _():
        mrun[...] = jnp.full((N, 1), NEG, jnp.float32)
        zrun[...] = jnp.zeros((N, 1), jnp.float32)

    col = j * VC + lax.broadcasted_iota(jnp.int32, (N, VC), 1)
    x = jnp.where(col < V, lg_ref[...], NEG)
    mold = mrun[...]
    mnew = jnp.maximum(mold, jnp.max(x, axis=1, keepdims=True))
    zrun[...] = zrun[...] * jnp.exp(mold - mnew) + jnp.sum(
        jnp.exp(x - mnew), axis=1, keepdims=True)
    mrun[...] = mnew

    @pl.when(j == NVC - 1)
    def _():
        m_out[...] = mrun[...]
        z_out[...] = zrun[...]


def _vstats(lg):
    return pl.pallas_call(
        _vstats_body,
        grid=(NVC,),
        in_specs=[pl.BlockSpec((N, VC), lambda j: (0, j))],
        out_specs=[pl.BlockSpec((N, 1), lambda j: (0, 0)),
                   pl.BlockSpec((N, 1), lambda j: (0, 0))],
        out_shape=[jax.ShapeDtypeStruct((N, 1), jnp.float32),
                   jax.ShapeDtypeStruct((N, 1), jnp.float32)],
        scratch_shapes=[pltpu.VMEM((N, 1), jnp.float32),
                        pltpu.VMEM((N, 1), jnp.float32)],
    )(lg)


def _mmgm_body(h_ref, k_ref, sims_out, gid_ref, gtmp, gm3):
    j = pl.program_id(0)
    sims = lax.dot_general(h_ref[...].astype(jnp.bfloat16),
                           k_ref[...].astype(jnp.bfloat16),
                           (((1,), (1,)), ((), ())),
                           preferred_element_type=jnp.float32)
    sims_out[...] = sims
    for c in range(KC // 128):
        gtmp[:, c:c + 1] = jnp.max(sims[:, c * 128:(c + 1) * 128], axis=1,
                                   keepdims=True)
    gm3[j] = gtmp[...]

    @pl.when(j == NKC - 1)
    def _():
        gid_ref[...] = jnp.zeros((N, 64), jnp.int32)
        pj = lax.broadcasted_iota(jnp.int32, (NKC, N, 16), 0)
        pc = lax.broadcasted_iota(jnp.int32, (NKC, N, 16), 2)
        pos3 = pj * 16 + pc
        for t in range(TOPK):
            v = gm3[...]
            mx = jnp.max(jnp.max(v, axis=0), axis=1)[None, :, None]
            pos = jnp.min(jnp.min(jnp.where(v >= mx, pos3, 1 << 30), axis=0),
                          axis=1)[None, :, None]
            gm3[...] = jnp.where(pos3 == pos, NEG, v)
            gid_ref[:, t:t + 1] = pos[0]


def _mmgm(h, keys):
    return pl.pallas_call(
        _mmgm_body,
        grid=(NKC,),
        in_specs=[pl.BlockSpec((N, D), lambda j: (0, 0)),
                  pl.BlockSpec((KC, D), lambda j: (j, 0))],
        out_specs=[pl.BlockSpec((N, KC), lambda j: (0, j)),
                   pl.BlockSpec((N, 64), lambda j: (0, 0))],
        out_shape=[jax.ShapeDtypeStruct((N, K_DB), jnp.float32),
                   jax.ShapeDtypeStruct((N, 64), jnp.int32)],
        scratch_shapes=[pltpu.VMEM((N, 16), jnp.float32),
                        pltpu.VMEM((NKC, N, 16), jnp.float32)],
    )(h, keys)


def _topk_cand_body(cand_ref, gid_in, ov_ref, oi_ref, cbuf):
    cbuf[...] = cand_ref[...]
    lane = lax.broadcasted_iota(jnp.int32, (N, TOPK * 128), 1)
    lane64 = lax.broadcasted_iota(jnp.int32, (N, 64), 1)
    gids = gid_in[...]
    ov_ref[...] = jnp.full((N, 64), NEG, jnp.float32)
    oi_ref[...] = jnp.zeros((N, 64), jnp.int32)
    for t in range(TOPK):
        v = cbuf[...]
        mx = jnp.max(v, axis=1, keepdims=True)
        pos = jnp.min(jnp.where(v >= mx, lane, 1 << 30), axis=1,
                      keepdims=True)
        cbuf[...] = jnp.where(lane == pos, NEG, v)
        seg = lax.shift_right_logical(pos, 7)
        off = lax.bitwise_and(pos, 127)
        gid = jnp.max(jnp.where(lane64 == seg, gids, 0), axis=1,
                      keepdims=True)
        ov_ref[:, t:t + 1] = mx
        oi_ref[:, t:t + 1] = gid * 128 + off


def _topk_cand(cand, gids):
    return pl.pallas_call(
        _topk_cand_body,
        out_shape=[jax.ShapeDtypeStruct((N, 64), jnp.float32),
                   jax.ShapeDtypeStruct((N, 64), jnp.int32)],
        scratch_shapes=[pltpu.VMEM((N, TOPK * 128), jnp.float32)],
    )(cand, gids)


def _sc_mesh():
    return plsc.VectorSubcoreMesh(core_axis_name="c", subcore_axis_name="s",
                                  num_cores=2, num_subcores=16)


_NW = 32
_RPW = (N * TOPK) // _NW


def _sc_seggather_body(table_hbm, gid_hbm, cand_hbm,
                       sel_v, ridx_v, rows_v, sem):
    wid = lax.axis_index("s") * 2 + lax.axis_index("c")
    for r in range(_ROWS_PW):
        n = wid * _ROWS_PW + r
        pltpu.sync_copy(gid_hbm.at[pl.ds(n * 64, 64)], sel_v)
        for c2 in range(2):
            g16 = sel_v[pl.ds(c2 * 16, 16)]
            ridx_v[pl.ds(c2 * 16, 16)] = n * NG + g16
        pltpu.async_copy(table_hbm.at[ridx_v], rows_v, sem).wait()
        pltpu.sync_copy(rows_v, cand_hbm.at[pl.ds(n * TOPK, TOPK), :])


def _sc_seggather(sims_table, gid_flat):
    return pl.kernel(
        _sc_seggather_body,
        out_type=jax.ShapeDtypeStruct((N * TOPK, 128), jnp.float32),
        mesh=_sc_mesh(),
        compiler_params=pltpu.CompilerParams(needs_layout_passes=False),
        scratch_types=[pltpu.VMEM((64,), jnp.int32),
                       pltpu.VMEM((TOPK,), jnp.int32),
                       pltpu.VMEM((TOPK, 128), jnp.float32),
                       pltpu.SemaphoreType.DMA],
    )(sims_table, gid_flat)


def _sc_gather_body(keys_hbm, kt_hbm, lgf_hbm, idx_hbm,
                    gath_hbm, toks_hbm, lgg_hbm,
                    idx_v, tokt_v, rows_v, tok_v, p_v, lgg_v, sem):
    wid = lax.axis_index("s") * 2 + lax.axis_index("c")
    base = wid * _RPW
    pltpu.sync_copy(idx_hbm.at[pl.ds(base, _RPW)], idx_v)
    pltpu.sync_copy(kt_hbm, tokt_v)
    lane = lax.iota(jnp.int32, 16)
    for c in range(_RPW // 16):
        iv = idx_v[pl.ds(c * 16, 16)]
        tk = plsc.load_gather(tokt_v, [iv])
        tok_v[pl.ds(c * 16, 16)] = tk
        n = (base + c * 16 + lane) >> 5
        p_v[pl.ds(c * 16, 16)] = n * V + tk
    pltpu.sync_copy(tok_v, toks_hbm.at[pl.ds(base, _RPW)])
    pltpu.async_copy(lgf_hbm.at[p_v], lgg_v, sem).wait()
    pltpu.sync_copy(lgg_v, lgg_hbm.at[pl.ds(base, _RPW)])
    for c in range(4):
        pltpu.async_copy(keys_hbm.at[idx_v.at[pl.ds(c * 32, 32)]],
                         rows_v, sem).wait()
        pltpu.sync_copy(rows_v, gath_hbm.at[pl.ds(base + c * 32, 32), :])


def _sc_gather(keys, key_tokens, lg_flat, idx_flat):
    return pl.kernel(
        _sc_gather_body,
        out_type=[jax.ShapeDtypeStruct((N * TOPK, D), jnp.float32),
                  jax.ShapeDtypeStruct((N * TOPK,), jnp.int32),
                  jax.ShapeDtypeStruct((N * TOPK,), jnp.float32)],
        mesh=_sc_mesh(),
        compiler_params=pltpu.CompilerParams(needs_layout_passes=False),
        scratch_types=[pltpu.VMEM((_RPW,), jnp.int32),
                       pltpu.VMEM((K_DB,), jnp.int32),
                       pltpu.VMEM((32, D), jnp.float32),
                       pltpu.VMEM((_RPW,), jnp.int32),
                       pltpu.VMEM((_RPW,), jnp.int32),
                       pltpu.VMEM((_RPW,), jnp.float32),
                       pltpu.SemaphoreType.DMA],
    )(keys, key_tokens, lg_flat, idx_flat)


def _mlp_body(h_ref, g_ref, tok_ref, lgg_ref, m_ref, z_ref,
              wbw_ref, bbw_ref, w1_ref, b1_ref, w2_ref, b2_ref,
              shift_out, corr_out):
    h = h_ref[...]
    g = g_ref[...]
    mean = jnp.mean(g, axis=1)
    bwlin = (jnp.dot(h, wbw_ref[:D, :], preferred_element_type=jnp.float32)
             + jnp.dot(mean, wbw_ref[D:, :], preferred_element_type=jnp.float32)
             + bbw_ref[0, 0])
    bw = jax.nn.sigmoid(bwlin)
    dist = jnp.sum(g * h[:, None, :], axis=2)
    scores = dist * bw
    mx = jnp.max(scores, axis=1, keepdims=True)
    e = jnp.exp(scores - mx)
    sp32 = e / jnp.sum(e, axis=1, keepdims=True)
    merged = jnp.sum(g * sp32[:, :, None], axis=1)
    hm = jax.nn.relu(
        jnp.dot(h, w1_ref[:D, :], preferred_element_type=jnp.float32)
        + jnp.dot(merged, w1_ref[D:, :], preferred_element_type=jnp.float32)
        + b1_ref[...])
    w = jax.nn.sigmoid(
        jnp.dot(hm, w2_ref[...], preferred_element_type=jnp.float32)
        + b2_ref[0, 0])
    tok = tok_ref[...]
    eq = tok[:, :, None] == tok[:, None, :]
    c = jnp.sum(jnp.where(eq, sp32[:, None, :], 0.0), axis=2)
    shift = m_ref[...] + jnp.log(z_ref[...]) - jnp.log(1.0 - w)
    shift_out[...] = shift
    corr_out[...] = jnp.log(jnp.exp(lgg_ref[...] - shift) + w * c)


def _mlp(h, g, toks, lgg, m, zs, W_bw, b_bw, W1, b1, W2, b2):
    return pl.pallas_call(
        _mlp_body,
        out_shape=[jax.ShapeDtypeStruct((N, 1), jnp.float32),
                   jax.ShapeDtypeStruct((N, TOPK), jnp.float32)],
    )(h, g, toks, lgg, m, zs, W_bw, b_bw, W1, b1, W2, b2)


_ROWS_PW = N // _NW


def _sc_vpass_body(lg_hbm, shift_hbm, corr_hbm, tok_hbm, out_hbm,
                   buf, shift_v, corr_v, tok_v):
    wid = lax.axis_index("s") * 2 + lax.axis_index("c")
    pltpu.sync_copy(shift_hbm, shift_v)
    for r in range(_ROWS_PW):
        n = wid * _ROWS_PW + r
        pltpu.sync_copy(lg_hbm.at[n], buf)
        pltpu.sync_copy(corr_hbm.at[pl.ds(n * TOPK, TOPK)], corr_v)
        pltpu.sync_copy(tok_hbm.at[pl.ds(n * TOPK, TOPK)], tok_v)
        sh = plsc.load_gather(shift_v, [jnp.full((16,), n, jnp.int32)])

        def body(i, _):
            s0 = i * 160
            for u in range(10):
                s = s0 + u * 16
                buf[pl.ds(s, 16)] = buf[pl.ds(s, 16)] - sh
            return 0

        lax.fori_loop(0, V // 160, body, 0)
        for ci in range(2):
            tk = tok_v[pl.ds(ci * 16, 16)]
            cr = corr_v[pl.ds(ci * 16, 16)]
            plsc.store_scatter(buf, [tk], cr)
        pltpu.sync_copy(buf, out_hbm.at[n])


def _sc_vpass(lg, shift, corr, toks):
    return pl.kernel(
        _sc_vpass_body,
        out_type=jax.ShapeDtypeStruct((N, V), jnp.float32),
        mesh=_sc_mesh(),
        compiler_params=pltpu.CompilerParams(needs_layout_passes=False),
        scratch_types=[pltpu.VMEM((V,), jnp.float32),
                       pltpu.VMEM((N,), jnp.float32),
                       pltpu.VMEM((TOPK,), jnp.float32),
                       pltpu.VMEM((TOPK,), jnp.int32)],
    )(lg, shift, corr, toks)


def kernel(hidden, logits, keys, key_tokens, W_bw, b_bw, W1, b1, W2, b2):
    h = hidden.reshape(N, D)
    lg = logits.reshape(N, V)
    m, zs = _vstats(lg)
    sims, gids = _mmgm(h, keys)
    cand = _sc_seggather(sims.reshape(N * NG, 128), gids.reshape(N * 64))
    _, tidx = _topk_cand(cand.reshape(N, TOPK * 128), gids)
    idx_flat = tidx[:, :TOPK].reshape(N * TOPK)
    gath, toks, lgg = _sc_gather(keys, key_tokens.astype(jnp.int32),
                                 lg.reshape(-1), idx_flat)
    shift, corr = _mlp(h, gath.reshape(N, TOPK, D),
                       toks.reshape(N, TOPK), lgg.reshape(N, TOPK), m, zs,
                       W_bw, b_bw.reshape(1, 1), W1, b1.reshape(1, D),
                       W2, b2.reshape(1, 1))
    out = _sc_vpass(lg, shift.reshape(N), corr.reshape(N * TOPK), toks)
    return out.reshape(B, T, V)


# speedup vs baseline: 1.0948x; 1.0948x over previous
import functools

import jax
import jax.numpy as jnp
from jax import lax
from jax.experimental import pallas as pl
from jax.experimental.pallas import tpu as pltpu
from jax.experimental.pallas import tpu_sc as plsc

B, T, D, V, K_DB, TOPK = 16, 8, 1024, 100000, 65536, 32
N = B * T
KC = 2048
NKC = K_DB // KC
NG = K_DB // 128
VC = 8192
NVC = (V + VC - 1) // VC
NEG = float("-inf")


def _vstats_body(lg_ref, m_out, z_out, mrun, zrun):
    j = pl.program_id(0)

    @pl.when(j == 0)
    def _():
        mrun[...] = jnp.full((N, 1), NEG, jnp.float32)
        zrun[...] = jnp.zeros((N, 1), jnp.float32)

    col = j * VC + lax.broadcasted_iota(jnp.int32, (N, VC), 1)
    x = jnp.where(col < V, lg_ref[...], NEG)
    mold = mrun[...]
    mnew = jnp.maximum(mold, jnp.max(x, axis=1, keepdims=True))
    zrun[...] = zrun[...] * jnp.exp(mold - mnew) + jnp.sum(
        jnp.exp(x - mnew), axis=1, keepdims=True)
    mrun[...] = mnew

    @pl.when(j == NVC - 1)
    def _():
        m_out[...] = mrun[...]
        z_out[...] = zrun[...]


def _vstats(lg):
    return pl.pallas_call(
        _vstats_body,
        grid=(NVC,),
        in_specs=[pl.BlockSpec((N, VC), lambda j: (0, j))],
        out_specs=[pl.BlockSpec((N, 1), lambda j: (0, 0)),
                   pl.BlockSpec((N, 1), lambda j: (0, 0))],
        out_shape=[jax.ShapeDtypeStruct((N, 1), jnp.float32),
                   jax.ShapeDtypeStruct((N, 1), jnp.float32)],
        scratch_shapes=[pltpu.VMEM((N, 1), jnp.float32),
                        pltpu.VMEM((N, 1), jnp.float32)],
    )(lg)


def _mmgm_body(h_ref, k_ref, sims_out, gid_ref, gtmp, gm3):
    j = pl.program_id(0)
    sims = lax.dot_general(h_ref[...].astype(jnp.bfloat16),
                           k_ref[...].astype(jnp.bfloat16),
                           (((1,), (1,)), ((), ())),
                           preferred_element_type=jnp.float32)
    sims_out[...] = sims
    for c in range(KC // 128):
        gtmp[:, c:c + 1] = jnp.max(sims[:, c * 128:(c + 1) * 128], axis=1,
                                   keepdims=True)
    gm3[j] = gtmp[...]

    @pl.when(j == NKC - 1)
    def _():
        gid_ref[...] = jnp.zeros((N, 64), jnp.int32)
        pj = lax.broadcasted_iota(jnp.int32, (NKC, N, 16), 0)
        pc = lax.broadcasted_iota(jnp.int32, (NKC, N, 16), 2)
        pos3 = pj * 16 + pc
        for t in range(TOPK):
            v = gm3[...]
            mx = jnp.max(jnp.max(v, axis=0), axis=1)[None, :, None]
            pos = jnp.min(jnp.min(jnp.where(v >= mx, pos3, 1 << 30), axis=0),
                          axis=1)[None, :, None]
            gm3[...] = jnp.where(pos3 == pos, NEG, v)
            gid_ref[:, t:t + 1] = pos[0]


def _mmgm(h, keys):
    return pl.pallas_call(
        _mmgm_body,
        grid=(NKC,),
        in_specs=[pl.BlockSpec((N, D), lambda j: (0, 0)),
                  pl.BlockSpec((KC, D), lambda j: (j, 0))],
        out_specs=[pl.BlockSpec((N, KC), lambda j: (0, j)),
                   pl.BlockSpec((N, 64), lambda j: (0, 0))],
        out_shape=[jax.ShapeDtypeStruct((N, K_DB), jnp.float32),
                   jax.ShapeDtypeStruct((N, 64), jnp.int32)],
        scratch_shapes=[pltpu.VMEM((N, 16), jnp.float32),
                        pltpu.VMEM((NKC, N, 16), jnp.float32)],
    )(h, keys)


def _topk_cand_body(cand_ref, gid_in, ov_ref, oi_ref, cbuf):
    cbuf[...] = cand_ref[...]
    lane = lax.broadcasted_iota(jnp.int32, (N, TOPK * 128), 1)
    lane64 = lax.broadcasted_iota(jnp.int32, (N, 64), 1)
    gids = gid_in[...]
    ov_ref[...] = jnp.full((N, 64), NEG, jnp.float32)
    oi_ref[...] = jnp.zeros((N, 64), jnp.int32)
    for t in range(TOPK):
        v = cbuf[...]
        mx = jnp.max(v, axis=1, keepdims=True)
        pos = jnp.min(jnp.where(v >= mx, lane, 1 << 30), axis=1,
                      keepdims=True)
        cbuf[...] = jnp.where(lane == pos, NEG, v)
        seg = lax.shift_right_logical(pos, 7)
        off = lax.bitwise_and(pos, 127)
        gid = jnp.max(jnp.where(lane64 == seg, gids, 0), axis=1,
                      keepdims=True)
        ov_ref[:, t:t + 1] = mx
        oi_ref[:, t:t + 1] = gid * 128 + off


def _topk_cand(cand, gids):
    return pl.pallas_call(
        _topk_cand_body,
        out_shape=[jax.ShapeDtypeStruct((N, 64), jnp.float32),
                   jax.ShapeDtypeStruct((N, 64), jnp.int32)],
        scratch_shapes=[pltpu.VMEM((N, TOPK * 128), jnp.float32)],
    )(cand, gids)


def _sc_mesh():
    return plsc.VectorSubcoreMesh(core_axis_name="c", subcore_axis_name="s",
                                  num_cores=2, num_subcores=16)


_NW = 32
_RPW = (N * TOPK) // _NW
_ROWS_PW = N // _NW


def _sc_seggather_body(table_hbm, gid_hbm, cand_hbm,
                       sel_v, ridx_v, rows_v, sem):
    wid = lax.axis_index("s") * 2 + lax.axis_index("c")
    for r in range(_ROWS_PW):
        n = wid * _ROWS_PW + r
        pltpu.sync_copy(gid_hbm.at[pl.ds(n * 64, 64)], sel_v)
        for c2 in range(2):
            g16 = sel_v[pl.ds(c2 * 16, 16)]
            ridx_v[pl.ds(c2 * 16, 16)] = n * NG + g16
        pltpu.async_copy(table_hbm.at[ridx_v], rows_v, sem).wait()
        pltpu.sync_copy(rows_v, cand_hbm.at[pl.ds(n * TOPK, TOPK), :])


def _sc_seggather(sims_table, gid_flat):
    return pl.kernel(
        _sc_seggather_body,
        out_type=jax.ShapeDtypeStruct((N * TOPK, 128), jnp.float32),
        mesh=_sc_mesh(),
        compiler_params=pltpu.CompilerParams(needs_layout_passes=False),
        scratch_types=[pltpu.VMEM((64,), jnp.int32),
                       pltpu.VMEM((TOPK,), jnp.int32),
                       pltpu.VMEM((TOPK, 128), jnp.float32),
                       pltpu.SemaphoreType.DMA],
    )(sims_table, gid_flat)


def _sc_gather_body(keys_hbm, kt_hbm, lgf_hbm, idx_hbm,
                    gath_hbm, toks_hbm, lgg_hbm,
                    idx_v, tokt_v, rows_v, tok_v, p_v, lgg_v, sem):
    wid = lax.axis_index("s") * 2 + lax.axis_index("c")
    base = wid * _RPW
    pltpu.sync_copy(idx_hbm.at[pl.ds(base, _RPW)], idx_v)
    pltpu.sync_copy(kt_hbm, tokt_v)
    lane = lax.iota(jnp.int32, 16)
    for c in range(_RPW // 16):
        iv = idx_v[pl.ds(c * 16, 16)]
        tk = plsc.load_gather(tokt_v, [iv])
        tok_v[pl.ds(c * 16, 16)] = tk
        n = (base + c * 16 + lane) >> 5
        p_v[pl.ds(c * 16, 16)] = n * V + tk
    pltpu.sync_copy(tok_v, toks_hbm.at[pl.ds(base, _RPW)])
    pltpu.async_copy(lgf_hbm.at[p_v], lgg_v, sem).wait()
    pltpu.sync_copy(lgg_v, lgg_hbm.at[pl.ds(base, _RPW)])
    for c in range(4):
        pltpu.async_copy(keys_hbm.at[idx_v.at[pl.ds(c * 32, 32)]],
                         rows_v, sem).wait()
        pltpu.sync_copy(rows_v, gath_hbm.at[pl.ds(base + c * 32, 32), :])


def _sc_gather(keys, key_tokens, lg_flat, idx_flat):
    return pl.kernel(
        _sc_gather_body,
        out_type=[jax.ShapeDtypeStruct((N * TOPK, D), jnp.float32),
                  jax.ShapeDtypeStruct((N * TOPK,), jnp.int32),
                  jax.ShapeDtypeStruct((N * TOPK,), jnp.float32)],
        mesh=_sc_mesh(),
        compiler_params=pltpu.CompilerParams(needs_layout_passes=False),
        scratch_types=[pltpu.VMEM((_RPW,), jnp.int32),
                       pltpu.VMEM((K_DB,), jnp.int32),
                       pltpu.VMEM((32, D), jnp.float32),
                       pltpu.VMEM((_RPW,), jnp.int32),
                       pltpu.VMEM((_RPW,), jnp.int32),
                       pltpu.VMEM((_RPW,), jnp.float32),
                       pltpu.SemaphoreType.DMA],
    )(keys, key_tokens, lg_flat, idx_flat)


def _mlp_body(h_ref, g_ref, tok_ref, lgg_ref, m_ref, z_ref,
              wbw_ref, bbw_ref, w1_ref, b1_ref, w2_ref, b2_ref,
              shift_out, corr_out):
    h = h_ref[...]
    g = g_ref[...]
    mean = jnp.mean(g, axis=1)
    bwlin = (jnp.dot(h, wbw_ref[:D, :], preferred_element_type=jnp.float32)
             + jnp.dot(mean, wbw_ref[D:, :], preferred_element_type=jnp.float32)
             + bbw_ref[0, 0])
    bw = jax.nn.sigmoid(bwlin)
    dist = jnp.sum(g * h[:, None, :], axis=2)
    scores = dist * bw
    mx = jnp.max(scores, axis=1, keepdims=True)
    e = jnp.exp(scores - mx)
    sp32 = e / jnp.sum(e, axis=1, keepdims=True)
    merged = jnp.sum(g * sp32[:, :, None], axis=1)
    hm = jax.nn.relu(
        jnp.dot(h, w1_ref[:D, :], preferred_element_type=jnp.float32)
        + jnp.dot(merged, w1_ref[D:, :], preferred_element_type=jnp.float32)
        + b1_ref[...])
    w = jax.nn.sigmoid(
        jnp.dot(hm, w2_ref[...], preferred_element_type=jnp.float32)
        + b2_ref[0, 0])
    tok = tok_ref[...]
    eq = tok[:, :, None] == tok[:, None, :]
    c = jnp.sum(jnp.where(eq, sp32[:, None, :], 0.0), axis=2)
    shift = m_ref[...] + jnp.log(z_ref[...]) - jnp.log(1.0 - w)
    shift_out[...] = shift
    corr_out[...] = jnp.log(jnp.exp(lgg_ref[...] - shift) + w * c)


def _mlp(h, g, toks, lgg, m, zs, W_bw, b_bw, W1, b1, W2, b2):
    return pl.pallas_call(
        _mlp_body,
        out_shape=[jax.ShapeDtypeStruct((N, 1), jnp.float32),
                   jax.ShapeDtypeStruct((N, TOPK), jnp.float32)],
    )(h, g, toks, lgg, m, zs, W_bw, b_bw, W1, b1, W2, b2)


RPS = 8
NRS = N // RPS


def _vpatch_body(lg_ref, shift_ref, tok_ref, corr_ref, out_ref):
    j = pl.program_id(0)
    out_ref[...] = lg_ref[...] - shift_ref[...]
    lanei = lax.broadcasted_iota(jnp.int32, (1, 160), 1)
    for r in range(RPS):
        for k in range(TOPK):
            t = tok_ref[j * RPS + r, k]
            c = corr_ref[j * RPS + r, k]
            base = pl.multiple_of(jnp.minimum(t >> 7, V // 128 - 1) * 128,
                                  128)
            off = t - base
            seg = out_ref[r:r + 1, pl.ds(base, 160)]
            out_ref[r:r + 1, pl.ds(base, 160)] = jnp.where(lanei == off,
                                                           c, seg)


def _vpatch(lg, shift, tok, corr):
    return pl.pallas_call(
        _vpatch_body,
        grid=(NRS,),
        in_specs=[pl.BlockSpec((RPS, V), lambda j: (j, 0)),
                  pl.BlockSpec((RPS, 1), lambda j: (j, 0)),
                  pl.BlockSpec(memory_space=pltpu.SMEM),
                  pl.BlockSpec(memory_space=pltpu.SMEM)],
        out_specs=pl.BlockSpec((RPS, V), lambda j: (j, 0)),
        out_shape=jax.ShapeDtypeStruct((N, V), jnp.float32),
    )(lg, shift, tok, corr)


def kernel(hidden, logits, keys, key_tokens, W_bw, b_bw, W1, b1, W2, b2):
    h = hidden.reshape(N, D)
    lg = logits.reshape(N, V)
    m, zs = _vstats(lg)
    sims, gids = _mmgm(h, keys)
    cand = _sc_seggather(sims.reshape(N * NG, 128), gids.reshape(N * 64))
    _, tidx = _topk_cand(cand.reshape(N, TOPK * 128), gids)
    idx_flat = tidx[:, :TOPK].reshape(N * TOPK)
    gath, toks, lgg = _sc_gather(keys, key_tokens.astype(jnp.int32),
                                 lg.reshape(-1), idx_flat)
    shift, corr = _mlp(h, gath.reshape(N, TOPK, D),
                       toks.reshape(N, TOPK), lgg.reshape(N, TOPK), m, zs,
                       W_bw, b_bw.reshape(1, 1), W1, b1.reshape(1, D),
                       W2, b2.reshape(1, 1))
    out = _vpatch(lg, shift, toks.reshape(N, TOPK), corr)
    return out.reshape(B, T, V)


# speedup vs baseline: 1.1702x; 1.0688x over previous
import functools

import jax
import jax.numpy as jnp
from jax import lax
from jax.experimental import pallas as pl
from jax.experimental.pallas import tpu as pltpu
from jax.experimental.pallas import tpu_sc as plsc

B, T, D, V, K_DB, TOPK = 16, 8, 1024, 100000, 65536, 32
N = B * T
KC = 4096
GPC = KC // 128
NKC = K_DB // KC
NG = K_DB // 128
VC = 8192
NVC = (V + VC - 1) // VC
NEG = float("-inf")


def _vstats_body(lg_ref, m_out, z_out, mrun, zrun):
    j = pl.program_id(0)

    @pl.when(j == 0)
    def _():
        mrun[...] = jnp.full((N, 1), NEG, jnp.float32)
        zrun[...] = jnp.zeros((N, 1), jnp.float32)

    col = j * VC + lax.broadcasted_iota(jnp.int32, (N, VC), 1)
    x = jnp.where(col < V, lg_ref[...], NEG)
    mold = mrun[...]
    mnew = jnp.maximum(mold, jnp.max(x, axis=1, keepdims=True))
    zrun[...] = zrun[...] * jnp.exp(mold - mnew) + jnp.sum(
        jnp.exp(x - mnew), axis=1, keepdims=True)
    mrun[...] = mnew

    @pl.when(j == NVC - 1)
    def _():
        m_out[...] = mrun[...]
        z_out[...] = zrun[...]


def _vstats(lg):
    return pl.pallas_call(
        _vstats_body,
        grid=(NVC,),
        in_specs=[pl.BlockSpec((N, VC), lambda j: (0, j))],
        out_specs=[pl.BlockSpec((N, 1), lambda j: (0, 0)),
                   pl.BlockSpec((N, 1), lambda j: (0, 0))],
        out_shape=[jax.ShapeDtypeStruct((N, 1), jnp.float32),
                   jax.ShapeDtypeStruct((N, 1), jnp.float32)],
        scratch_shapes=[pltpu.VMEM((N, 1), jnp.float32),
                        pltpu.VMEM((N, 1), jnp.float32)],
    )(lg)


def _mmgm_body(h_ref, k_ref, sims_out, gid_ref, gtmp, gm3):
    j = pl.program_id(0)
    sims = lax.dot_general(h_ref[...].astype(jnp.bfloat16),
                           k_ref[...].astype(jnp.bfloat16),
                           (((1,), (1,)), ((), ())),
                           preferred_element_type=jnp.float32)
    sims_out[...] = sims
    for c in range(GPC):
        gtmp[:, c:c + 1] = jnp.max(sims[:, c * 128:(c + 1) * 128], axis=1,
                                   keepdims=True)
    gm3[j] = gtmp[...]

    @pl.when(j == NKC - 1)
    def _():
        gid_ref[...] = jnp.zeros((N, 64), jnp.int32)
        pj = lax.broadcasted_iota(jnp.int32, (NKC, N, GPC), 0)
        pc = lax.broadcasted_iota(jnp.int32, (NKC, N, GPC), 2)
        pos3 = pj * GPC + pc
        for t in range(TOPK):
            v = gm3[...]
            mx = jnp.max(jnp.max(v, axis=0), axis=1)[None, :, None]
            pos = jnp.min(jnp.min(jnp.where(v >= mx, pos3, 1 << 30), axis=0),
                          axis=1)[None, :, None]
            gm3[...] = jnp.where(pos3 == pos, NEG, v)
            gid_ref[:, t:t + 1] = pos[0]


def _mmgm(h, keys):
    return pl.pallas_call(
        _mmgm_body,
        grid=(NKC,),
        in_specs=[pl.BlockSpec((N, D), lambda j: (0, 0)),
                  pl.BlockSpec((KC, D), lambda j: (j, 0))],
        out_specs=[pl.BlockSpec((N, KC), lambda j: (0, j)),
                   pl.BlockSpec((N, 64), lambda j: (0, 0))],
        out_shape=[jax.ShapeDtypeStruct((N, K_DB), jnp.float32),
                   jax.ShapeDtypeStruct((N, 64), jnp.int32)],
        scratch_shapes=[pltpu.VMEM((N, GPC), jnp.float32),
                        pltpu.VMEM((NKC, N, GPC), jnp.float32)],
    )(h, keys)


def _topk_cand_body(cand_ref, gid_in, ov_ref, oi_ref, cbuf):
    cbuf[...] = cand_ref[...]
    lane = lax.broadcasted_iota(jnp.int32, (N, TOPK * 128), 1)
    lane64 = lax.broadcasted_iota(jnp.int32, (N, 64), 1)
    gids = gid_in[...]
    ov_ref[...] = jnp.full((N, 64), NEG, jnp.float32)
    oi_ref[...] = jnp.zeros((N, 64), jnp.int32)
    for t in range(TOPK):
        v = cbuf[...]
        mx = jnp.max(v, axis=1, keepdims=True)
        pos = jnp.min(jnp.where(v >= mx, lane, 1 << 30), axis=1,
                      keepdims=True)
        cbuf[...] = jnp.where(lane == pos, NEG, v)
        seg = lax.shift_right_logical(pos, 7)
        off = lax.bitwise_and(pos, 127)
        gid = jnp.max(jnp.where(lane64 == seg, gids, 0), axis=1,
                      keepdims=True)
        ov_ref[:, t:t + 1] = mx
        oi_ref[:, t:t + 1] = gid * 128 + off


def _topk_cand(cand, gids):
    return pl.pallas_call(
        _topk_cand_body,
        out_shape=[jax.ShapeDtypeStruct((N, 64), jnp.float32),
                   jax.ShapeDtypeStruct((N, 64), jnp.int32)],
        scratch_shapes=[pltpu.VMEM((N, TOPK * 128), jnp.float32)],
    )(cand, gids)


def _sc_mesh():
    return plsc.VectorSubcoreMesh(core_axis_name="c", subcore_axis_name="s",
                                  num_cores=2, num_subcores=16)


_NW = 32
_RPW = (N * TOPK) // _NW
_ROWS_PW = N // _NW


def _sc_seggather_body(table_hbm, gid_hbm, cand_hbm,
                       sel_v, ridx_v, rows_v, sem):
    wid = lax.axis_index("s") * 2 + lax.axis_index("c")
    for r in range(_ROWS_PW):
        n = wid * _ROWS_PW + r
        pltpu.sync_copy(gid_hbm.at[pl.ds(n * 64, 64)], sel_v)
        for c2 in range(2):
            g16 = sel_v[pl.ds(c2 * 16, 16)]
            ridx_v[pl.ds(c2 * 16, 16)] = n * NG + g16
        pltpu.async_copy(table_hbm.at[ridx_v], rows_v, sem).wait()
        pltpu.sync_copy(rows_v, cand_hbm.at[pl.ds(n * TOPK, TOPK), :])


def _sc_seggather(sims_table, gid_flat):
    return pl.kernel(
        _sc_seggather_body,
        out_type=jax.ShapeDtypeStruct((N * TOPK, 128), jnp.float32),
        mesh=_sc_mesh(),
        compiler_params=pltpu.CompilerParams(needs_layout_passes=False),
        scratch_types=[pltpu.VMEM((64,), jnp.int32),
                       pltpu.VMEM((TOPK,), jnp.int32),
                       pltpu.VMEM((TOPK, 128), jnp.float32),
                       pltpu.SemaphoreType.DMA],
    )(sims_table, gid_flat)


def _sc_gather_body(keys_hbm, kt_hbm, lgf_hbm, idx_hbm,
                    gath_hbm, toks_hbm, lgg_hbm,
                    idx_v, tokt_v, rows_v, tok_v, p_v, lgg_v, sem):
    wid = lax.axis_index("s") * 2 + lax.axis_index("c")
    base = wid * _RPW
    pltpu.sync_copy(idx_hbm.at[pl.ds(base, _RPW)], idx_v)
    pltpu.sync_copy(kt_hbm, tokt_v)
    lane = lax.iota(jnp.int32, 16)
    for c in range(_RPW // 16):
        iv = idx_v[pl.ds(c * 16, 16)]
        tk = plsc.load_gather(tokt_v, [iv])
        tok_v[pl.ds(c * 16, 16)] = tk
        n = (base + c * 16 + lane) >> 5
        p_v[pl.ds(c * 16, 16)] = n * V + tk
    pltpu.sync_copy(tok_v, toks_hbm.at[pl.ds(base, _RPW)])
    pltpu.async_copy(lgf_hbm.at[p_v], lgg_v, sem).wait()
    pltpu.sync_copy(lgg_v, lgg_hbm.at[pl.ds(base, _RPW)])
    for c in range(4):
        pltpu.async_copy(keys_hbm.at[idx_v.at[pl.ds(c * 32, 32)]],
                         rows_v, sem).wait()
        pltpu.sync_copy(rows_v, gath_hbm.at[pl.ds(base + c * 32, 32), :])


def _sc_gather(keys, key_tokens, lg_flat, idx_flat):
    return pl.kernel(
        _sc_gather_body,
        out_type=[jax.ShapeDtypeStruct((N * TOPK, D), jnp.float32),
                  jax.ShapeDtypeStruct((N * TOPK,), jnp.int32),
                  jax.ShapeDtypeStruct((N * TOPK,), jnp.float32)],
        mesh=_sc_mesh(),
        compiler_params=pltpu.CompilerParams(needs_layout_passes=False),
        scratch_types=[pltpu.VMEM((_RPW,), jnp.int32),
                       pltpu.VMEM((K_DB,), jnp.int32),
                       pltpu.VMEM((32, D), jnp.float32),
                       pltpu.VMEM((_RPW,), jnp.int32),
                       pltpu.VMEM((_RPW,), jnp.int32),
                       pltpu.VMEM((_RPW,), jnp.float32),
                       pltpu.SemaphoreType.DMA],
    )(keys, key_tokens, lg_flat, idx_flat)


def _mlp_body(h_ref, g_ref, tok_ref, lgg_ref, m_ref, z_ref,
              wbw_ref, bbw_ref, w1_ref, b1_ref, w2_ref, b2_ref,
              shift_out, corr_out):
    h = h_ref[...]
    g = g_ref[...]
    mean = jnp.mean(g, axis=1)
    bwlin = (jnp.dot(h, wbw_ref[:D, :], preferred_element_type=jnp.float32)
             + jnp.dot(mean, wbw_ref[D:, :], preferred_element_type=jnp.float32)
             + bbw_ref[0, 0])
    bw = jax.nn.sigmoid(bwlin)
    dist = jnp.sum(g * h[:, None, :], axis=2)
    scores = dist * bw
    mx = jnp.max(scores, axis=1, keepdims=True)
    e = jnp.exp(scores - mx)
    sp32 = e / jnp.sum(e, axis=1, keepdims=True)
    merged = jnp.sum(g * sp32[:, :, None], axis=1)
    hm = jax.nn.relu(
        jnp.dot(h, w1_ref[:D, :], preferred_element_type=jnp.float32)
        + jnp.dot(merged, w1_ref[D:, :], preferred_element_type=jnp.float32)
        + b1_ref[...])
    w = jax.nn.sigmoid(
        jnp.dot(hm, w2_ref[...], preferred_element_type=jnp.float32)
        + b2_ref[0, 0])
    tok = tok_ref[...]
    eq = tok[:, :, None] == tok[:, None, :]
    c = jnp.sum(jnp.where(eq, sp32[:, None, :], 0.0), axis=2)
    shift = m_ref[...] + jnp.log(z_ref[...]) - jnp.log(1.0 - w)
    shift_out[...] = shift
    corr_out[...] = jnp.log(jnp.exp(lgg_ref[...] - shift) + w * c)


def _mlp(h, g, toks, lgg, m, zs, W_bw, b_bw, W1, b1, W2, b2):
    return pl.pallas_call(
        _mlp_body,
        out_shape=[jax.ShapeDtypeStruct((N, 1), jnp.float32),
                   jax.ShapeDtypeStruct((N, TOPK), jnp.float32)],
    )(h, g, toks, lgg, m, zs, W_bw, b_bw, W1, b1, W2, b2)


RPS = 8
NRS = N // RPS


def _vpatch_body(lg_ref, shift_ref, tok_ref, corr_ref, out_ref):
    j = pl.program_id(0)
    out_ref[...] = lg_ref[...] - shift_ref[...]
    lanei = lax.broadcasted_iota(jnp.int32, (1, 160), 1)
    for r in range(RPS):
        for k in range(TOPK):
            t = tok_ref[j * RPS + r, k]
            c = corr_ref[j * RPS + r, k]
            base = pl.multiple_of(jnp.minimum(t >> 7, V // 128 - 1) * 128,
                                  128)
            off = t - base
            seg = out_ref[r:r + 1, pl.ds(base, 160)]
            out_ref[r:r + 1, pl.ds(base, 160)] = jnp.where(lanei == off,
                                                           c, seg)


def _vpatch(lg, shift, tok, corr):
    return pl.pallas_call(
        _vpatch_body,
        grid=(NRS,),
        in_specs=[pl.BlockSpec((RPS, V), lambda j: (j, 0)),
                  pl.BlockSpec((RPS, 1), lambda j: (j, 0)),
                  pl.BlockSpec(memory_space=pltpu.SMEM),
                  pl.BlockSpec(memory_space=pltpu.SMEM)],
        out_specs=pl.BlockSpec((RPS, V), lambda j: (j, 0)),
        out_shape=jax.ShapeDtypeStruct((N, V), jnp.float32),
    )(lg, shift, tok, corr)


def kernel(hidden, logits, keys, key_tokens, W_bw, b_bw, W1, b1, W2, b2):
    h = hidden.reshape(N, D)
    lg = logits.reshape(N, V)
    m, zs = _vstats(lg)
    sims, gids = _mmgm(h, keys)
    cand = _sc_seggather(sims.reshape(N * NG, 128), gids.reshape(N * 64))
    _, tidx = _topk_cand(cand.reshape(N, TOPK * 128), gids)
    idx_flat = tidx[:, :TOPK].reshape(N * TOPK)
    gath, toks, lgg = _sc_gather(keys, key_tokens.astype(jnp.int32),
                                 lg.reshape(-1), idx_flat)
    shift, corr = _mlp(h, gath.reshape(N, TOPK, D),
                       toks.reshape(N, TOPK), lgg.reshape(N, TOPK), m, zs,
                       W_bw, b_bw.reshape(1, 1), W1, b1.reshape(1, D),
                       W2, b2.reshape(1, 1))
    out = _vpatch(lg, shift, toks.reshape(N, TOPK), corr)
    return out.reshape(B, T, V)
